# baseline (speedup 1.0000x reference)
import functools

import jax
import jax.numpy as jnp
from jax import lax
from jax.experimental import pallas as pl
from jax.experimental.pallas import tpu as pltpu
from jax.experimental.pallas import tpu_sc as plsc

_NB = 4
_GAMMA = 0.95
_BS = 256
_M = 2048
_D = 64
_H = 1024
_G = 16
_SC_CORES = 2
_SC_SUBCORES = 16
_BIG = 3.0e38


def _topk4_sc(usage):
    nw = _SC_CORES * _SC_SUBCORES
    bpw = _BS // nw
    mesh = plsc.VectorSubcoreMesh(core_axis_name="c", subcore_axis_name="s")

    @functools.partial(
        pl.kernel,
        out_type=jax.ShapeDtypeStruct((_BS * _NB,), jnp.int32),
        mesh=mesh,
        scratch_types=[
            pltpu.VMEM((_M,), jnp.float32),
            pltpu.VMEM((bpw * _NB,), jnp.int32),
        ],
    )
    def topk_kernel(u_hbm, out_hbm, u_buf, out_buf):
        wid = lax.axis_index("s") * _SC_CORES + lax.axis_index("c")
        lanes = lax.broadcasted_iota(jnp.int32, (16,), 0)
        r0 = jnp.zeros((16,), jnp.int32)
        r1 = jnp.zeros((16,), jnp.int32)
        for bi in range(bpw):
            batch = wid * bpw + bi
            pltpu.sync_copy(u_hbm.at[batch], u_buf)
            for p in range(_NB):
                def body(j, c):
                    mv, mi = c
                    v = u_buf[pl.ds(pl.multiple_of(j * 16, 16), 16)]
                    fi = j * 16 + lanes
                    take = v < mv
                    return (jnp.where(take, v, mv), jnp.where(take, fi, mi))

                mv, mi = lax.fori_loop(
                    0, _M // 16, body,
                    (jnp.full((16,), _BIG, jnp.float32),
                     jnp.zeros((16,), jnp.int32)))
                m = mv[0]
                i = mi[0]
                for l in range(1, 16):
                    v = mv[l]
                    ii = mi[l]
                    upd = (v < m) | ((v == m) & (ii < i))
                    m = jnp.where(upd, v, m)
                    i = jnp.where(upd, ii, i)
                pos = bi * _NB + p
                if pos < 16:
                    r0 = jnp.where(lanes == pos, i, r0)
                else:
                    r1 = jnp.where(lanes == (pos - 16), i, r1)
                blk = (i // 16) * 16
                sl = pl.ds(pl.multiple_of(blk, 16), 16)
                u_buf[sl] = jnp.where(lanes == (i - blk), _BIG, u_buf[sl])
        out_buf[pl.ds(0, 16)] = r0
        out_buf[pl.ds(16, 16)] = r1
        pltpu.sync_copy(
            out_buf, out_hbm.at[pl.ds(wid * (bpw * _NB), bpw * _NB)])

    return topk_kernel(usage)


def _query_kernel(h_ref, w_ref, bias_ref, q_ref, qss_ref):
    q = lax.dot_general(h_ref[...], w_ref[...], (((1,), (1,)), ((), ())),
                        preferred_element_type=jnp.float32) + bias_ref[...]
    q_ref[...] = q
    qss_ref[...] = jnp.full((1, 128), jnp.sum(q * q) * (1.0 / 128.0))


def _query_call(hidden, W, bias2):
    return pl.pallas_call(
        _query_kernel,
        out_shape=[
            jax.ShapeDtypeStruct((_BS, _NB * _D), jnp.float32),
            jax.ShapeDtypeStruct((1, 128), jnp.float32),
        ],
    )(hidden, W, bias2)


def _stage1_kernel(mem_ref, rw_ref, alpha_ref, idx_ref, q_ref,
                   mnew_ref, logits_ref, wws_ref, mn2_ref):
    if True:
        mnew_ref[...] = mem_ref[...]
        logits_ref[...] = jnp.zeros_like(logits_ref)
        wws_ref[...] = jnp.zeros_like(wws_ref)
        mn2_ref[...] = jnp.zeros_like(mn2_ref)
        return
    for g in range(_G):
        a = jax.nn.sigmoid(alpha_ref[g, :, 0])
        ww = a[:, None] * rw_ref[g]
        col = idx_ref[g, 0, :]
        hit = lax.broadcasted_iota(jnp.int32, (_NB, _M), 1) == col[:, None]
        ww = ww + jnp.where(hit, (1.0 - a)[:, None], 0.0)
        q = q_ref[g]
        delta = lax.dot_general(ww, q, (((0,), (0,)), ((), ())),
                                preferred_element_type=jnp.float32)
        mn = mem_ref[g] + delta
        mnew_ref[g] = mn
        logits_ref[g] = lax.dot_general(q, mn, (((1,), (1,)), ((), ())),
                                        preferred_element_type=jnp.float32)
        wws_ref[g, 0, :] = jnp.sum(ww, axis=0)
        mn2_ref[g, 0, :] = jnp.full((128,), jnp.sum(mn * mn) * (1.0 / 128.0))


def _stage1_call(memory, read_weight, alpha, idx3, q3):
    return pl.pallas_call(
        _stage1_kernel,
        grid=(_BS // _G,),
        in_specs=[
            pl.BlockSpec((_G, _M, _D), lambda i: (i, 0, 0)),
            pl.BlockSpec((_G, _NB, _M), lambda i: (i, 0, 0)),
            pl.BlockSpec((_G, _NB, 1), lambda i: (i, 0, 0)),
            pl.BlockSpec((_G, 1, _NB), lambda i: (i, 0, 0)),
            pl.BlockSpec((_G, _NB, _D), lambda i: (i, 0, 0)),
        ],
        out_specs=[
            pl.BlockSpec((_G, _M, _D), lambda i: (i, 0, 0)),
            pl.BlockSpec((_G, _NB, _M), lambda i: (i, 0, 0)),
            pl.BlockSpec((_G, 1, _M), lambda i: (i, 0, 0)),
            pl.BlockSpec((_G, 1, 128), lambda i: (i, 0, 0)),
        ],
        out_shape=[
            jax.ShapeDtypeStruct((_BS, _M, _D), jnp.float32),
            jax.ShapeDtypeStruct((_BS, _NB, _M), jnp.float32),
            jax.ShapeDtypeStruct((_BS, 1, _M), jnp.float32),
            jax.ShapeDtypeStruct((_BS, 1, 128), jnp.float32),
        ],
    )(memory, read_weight, alpha, idx3, q3)


def _stage2_kernel(logits_ref, mnew_ref, usage_ref, wws_ref, mn2_ref, qss_ref,
                   rw_ref, rv_ref, uw_ref):
    scale = jnp.sqrt(jnp.sum(mn2_ref[...])) * jnp.sqrt(jnp.sum(qss_ref[...]))
    inv = 1.0 / scale
    for g in range(_G):
        l = logits_ref[g] * inv
        m = jnp.max(l, axis=1, keepdims=True)
        e = jnp.exp(l - m)
        r = e / jnp.sum(e, axis=1, keepdims=True)
        rw_ref[g] = r
        rv_ref[g] = lax.dot_general(r, mnew_ref[g], (((1,), (0,)), ((), ())),
                                    preferred_element_type=jnp.float32)
        uw_ref[g, 0, :] = (_GAMMA * usage_ref[g, 0, :] + jnp.sum(r, axis=0)
                           + wws_ref[g, 0, :])


def _stage2_call(logits, mnew, usage3, wws, mn2, qss):
    return pl.pallas_call(
        _stage2_kernel,
        grid=(_BS // _G,),
        in_specs=[
            pl.BlockSpec((_G, _NB, _M), lambda i: (i, 0, 0)),
            pl.BlockSpec((_G, _M, _D), lambda i: (i, 0, 0)),
            pl.BlockSpec((_G, 1, _M), lambda i: (i, 0, 0)),
            pl.BlockSpec((_G, 1, _M), lambda i: (i, 0, 0)),
            pl.BlockSpec((_BS, 1, 128), lambda i: (0, 0, 0)),
            pl.BlockSpec((1, 128), lambda i: (0, 0)),
        ],
        out_specs=[
            pl.BlockSpec((_G, _NB, _M), lambda i: (i, 0, 0)),
            pl.BlockSpec((_G, _NB, _D), lambda i: (i, 0, 0)),
            pl.BlockSpec((_G, 1, _M), lambda i: (i, 0, 0)),
        ],
        out_shape=[
            jax.ShapeDtypeStruct((_BS, _NB, _M), jnp.float32),
            jax.ShapeDtypeStruct((_BS, _NB, _D), jnp.float32),
            jax.ShapeDtypeStruct((_BS, 1, _M), jnp.float32),
        ],
    )(logits, mnew, usage3, wws, mn2, qss)


def _copy_kernel(x_ref, y_ref):
    y_ref[...] = x_ref[...]


def kernel(memory, hidden, read_weight, usage_weight, alpha, W, b):
    x = memory.reshape(32768, 1024)
    y = pl.pallas_call(
        _copy_kernel,
        grid=(16,),
        in_specs=[pl.BlockSpec((2048, 1024), lambda i: (i, 0))],
        out_specs=pl.BlockSpec((2048, 1024), lambda i: (i, 0)),
        out_shape=jax.ShapeDtypeStruct((32768, 1024), jnp.float32),
    )(x)
    return (y[:256, :256], y.reshape(_BS, _M, _D), y[:256, :256], y[:256, :2048])


def _kernel_real(memory, hidden, read_weight, usage_weight, alpha, W, b):
    idx = jnp.zeros((_BS * _NB,), jnp.int32)
    query, qss = _query_call(hidden, W, b.reshape(1, -1))
    idx3 = idx.reshape(_BS, 1, _NB)
    q3 = query.reshape(_BS, _NB, _D)
    mnew, logits, wws, mn2 = _stage1_call(memory, read_weight, alpha,
                                          idx3, q3)
    return (query.reshape(_BS, _NB * _D), mnew, logits, wws.reshape(_BS, _M))


# speedup vs baseline: 3.0450x; 3.0450x over previous
import functools

import jax
import jax.numpy as jnp
from jax import lax
from jax.experimental import pallas as pl
from jax.experimental.pallas import tpu as pltpu
from jax.experimental.pallas import tpu_sc as plsc

_NB = 4
_GAMMA = 0.95
_BS = 256
_M = 2048
_D = 64
_H = 1024
_G = 8
_SC_CORES = 2
_SC_SUBCORES = 16
_BIG = 3.0e38


def _topk4_sc(usage):
    nw = _SC_CORES * _SC_SUBCORES
    bpw = _BS // nw
    mesh = plsc.VectorSubcoreMesh(core_axis_name="c", subcore_axis_name="s")

    @functools.partial(
        pl.kernel,
        out_type=jax.ShapeDtypeStruct((_BS * _NB,), jnp.int32),
        mesh=mesh,
        scratch_types=[
            pltpu.VMEM((_M,), jnp.float32),
            pltpu.VMEM((bpw * _NB,), jnp.int32),
        ],
    )
    def topk_kernel(u_hbm, out_hbm, u_buf, out_buf):
        wid = lax.axis_index("s") * _SC_CORES + lax.axis_index("c")
        lanes = lax.broadcasted_iota(jnp.int32, (16,), 0)
        r0 = jnp.zeros((16,), jnp.int32)
        r1 = jnp.zeros((16,), jnp.int32)
        for bi in range(bpw):
            batch = wid * bpw + bi
            pltpu.sync_copy(u_hbm.at[batch], u_buf)
            for p in range(_NB):
                def body(j, c):
                    mv, mi = c
                    v = u_buf[pl.ds(pl.multiple_of(j * 16, 16), 16)]
                    fi = j * 16 + lanes
                    take = v < mv
                    return (jnp.where(take, v, mv), jnp.where(take, fi, mi))

                mv, mi = lax.fori_loop(
                    0, _M // 16, body,
                    (jnp.full((16,), _BIG, jnp.float32),
                     jnp.zeros((16,), jnp.int32)))
                m = mv[0]
                i = mi[0]
                for l in range(1, 16):
                    v = mv[l]
                    ii = mi[l]
                    upd = (v < m) | ((v == m) & (ii < i))
                    m = jnp.where(upd, v, m)
                    i = jnp.where(upd, ii, i)
                pos = bi * _NB + p
                if pos < 16:
                    r0 = jnp.where(lanes == pos, i, r0)
                else:
                    r1 = jnp.where(lanes == (pos - 16), i, r1)
                blk = (i // 16) * 16
                sl = pl.ds(pl.multiple_of(blk, 16), 16)
                u_buf[sl] = jnp.where(lanes == (i - blk), _BIG, u_buf[sl])
        out_buf[pl.ds(0, 16)] = r0
        out_buf[pl.ds(16, 16)] = r1
        pltpu.sync_copy(
            out_buf, out_hbm.at[pl.ds(wid * (bpw * _NB), bpw * _NB)])

    return topk_kernel(usage)


def _query_kernel(h_ref, w_ref, bias_ref, q_ref, qss_ref):
    q = lax.dot_general(h_ref[...], w_ref[...], (((1,), (1,)), ((), ())),
                        preferred_element_type=jnp.float32) + bias_ref[...]
    q_ref[...] = q
    qss_ref[...] = jnp.full((1, 128), jnp.sum(q * q) * (1.0 / 128.0))


def _query_call(hidden, W, bias2):
    return pl.pallas_call(
        _query_kernel,
        out_shape=[
            jax.ShapeDtypeStruct((_BS, _NB * _D), jnp.float32),
            jax.ShapeDtypeStruct((1, 128), jnp.float32),
        ],
    )(hidden, W, bias2)


def _stage1_kernel(memt_ref, rw_ref, alpha_ref, idx_ref, q_ref,
                   mnewt_ref, logits_ref, wws_ref, mn2_ref):
    for g in range(_G):
        a = jax.nn.sigmoid(alpha_ref[g, :, 0])
        ww = a[:, None] * rw_ref[g]
        col = idx_ref[g, 0, :]
        hit = lax.broadcasted_iota(jnp.int32, (_NB, _M), 1) == col[:, None]
        ww = ww + jnp.where(hit, (1.0 - a)[:, None], 0.0)
        q = q_ref[g]
        deltat = lax.dot_general(q, ww, (((0,), (0,)), ((), ())),
                                 preferred_element_type=jnp.float32)
        mnt = memt_ref[g] + deltat
        mnewt_ref[g] = mnt
        logits_ref[g] = lax.dot_general(q, mnt, (((1,), (0,)), ((), ())),
                                        preferred_element_type=jnp.float32)
        wws_ref[g, 0, :] = jnp.sum(ww, axis=0)
        mn2_ref[g, 0, :] = jnp.full((128,), jnp.sum(mnt * mnt) * (1.0 / 128.0))


def _stage1_call(memt, read_weight, alpha, idx3, q3):
    return pl.pallas_call(
        _stage1_kernel,
        grid=(_BS // _G,),
        in_specs=[
            pl.BlockSpec((_G, _D, _M), lambda i: (i, 0, 0)),
            pl.BlockSpec((_G, _NB, _M), lambda i: (i, 0, 0)),
            pl.BlockSpec((_G, _NB, 1), lambda i: (i, 0, 0)),
            pl.BlockSpec((_G, 1, _NB), lambda i: (i, 0, 0)),
            pl.BlockSpec((_G, _NB, _D), lambda i: (i, 0, 0)),
        ],
        out_specs=[
            pl.BlockSpec((_G, _D, _M), lambda i: (i, 0, 0)),
            pl.BlockSpec((_G, _NB, _M), lambda i: (i, 0, 0)),
            pl.BlockSpec((_G, 1, _M), lambda i: (i, 0, 0)),
            pl.BlockSpec((_G, 1, 128), lambda i: (i, 0, 0)),
        ],
        out_shape=[
            jax.ShapeDtypeStruct((_BS, _D, _M), jnp.float32),
            jax.ShapeDtypeStruct((_BS, _NB, _M), jnp.float32),
            jax.ShapeDtypeStruct((_BS, 1, _M), jnp.float32),
            jax.ShapeDtypeStruct((_BS, 1, 128), jnp.float32),
        ],
    )(memt, read_weight, alpha, idx3, q3)


def _stage2_kernel(logits_ref, mnewt_ref, usage_ref, wws_ref, mn2_ref, qss_ref,
                   rw_ref, rv_ref, uw_ref):
    scale = jnp.sqrt(jnp.sum(mn2_ref[...])) * jnp.sqrt(jnp.sum(qss_ref[...]))
    inv = 1.0 / scale
    for g in range(_G):
        l = logits_ref[g] * inv
        m = jnp.max(l, axis=1, keepdims=True)
        e = jnp.exp(l - m)
        r = e / jnp.sum(e, axis=1, keepdims=True)
        rw_ref[g] = r
        rv_ref[g] = lax.dot_general(r, mnewt_ref[g], (((1,), (1,)), ((), ())),
                                    preferred_element_type=jnp.float32)
        uw_ref[g, 0, :] = (_GAMMA * usage_ref[g, 0, :] + jnp.sum(r, axis=0)
                           + wws_ref[g, 0, :])


def _stage2_call(logits, mnewt, usage3, wws, mn2, qss):
    return pl.pallas_call(
        _stage2_kernel,
        grid=(_BS // _G,),
        in_specs=[
            pl.BlockSpec((_G, _NB, _M), lambda i: (i, 0, 0)),
            pl.BlockSpec((_G, _D, _M), lambda i: (i, 0, 0)),
            pl.BlockSpec((_G, 1, _M), lambda i: (i, 0, 0)),
            pl.BlockSpec((_G, 1, _M), lambda i: (i, 0, 0)),
            pl.BlockSpec((_BS, 1, 128), lambda i: (0, 0, 0)),
            pl.BlockSpec((1, 128), lambda i: (0, 0)),
        ],
        out_specs=[
            pl.BlockSpec((_G, _NB, _M), lambda i: (i, 0, 0)),
            pl.BlockSpec((_G, _NB, _D), lambda i: (i, 0, 0)),
            pl.BlockSpec((_G, 1, _M), lambda i: (i, 0, 0)),
        ],
        out_shape=[
            jax.ShapeDtypeStruct((_BS, _NB, _M), jnp.float32),
            jax.ShapeDtypeStruct((_BS, _NB, _D), jnp.float32),
            jax.ShapeDtypeStruct((_BS, 1, _M), jnp.float32),
        ],
    )(logits, mnewt, usage3, wws, mn2, qss)


def kernel(memory, hidden, read_weight, usage_weight, alpha, W, b):
    idx = _topk4_sc(usage_weight)
    query, qss = _query_call(hidden, W, b.reshape(1, -1))
    idx3 = idx.reshape(_BS, 1, _NB)
    q3 = query.reshape(_BS, _NB, _D)
    memt = jnp.transpose(memory, (0, 2, 1))
    mnewt, logits, wws, mn2 = _stage1_call(memt, read_weight, alpha,
                                           idx3, q3)
    rw, rv, uw = _stage2_call(logits, mnewt,
                              usage_weight.reshape(_BS, 1, _M), wws, mn2, qss)
    memory_new = jnp.transpose(mnewt, (0, 2, 1))
    return (rv.reshape(_BS, _NB * _D), memory_new, rw, uw.reshape(_BS, _M))


# speedup vs baseline: 3.3534x; 1.1013x over previous
import functools

import jax
import jax.numpy as jnp
from jax import lax
from jax.experimental import pallas as pl
from jax.experimental.pallas import tpu as pltpu
from jax.experimental.pallas import tpu_sc as plsc

_NB = 4
_GAMMA = 0.95
_BS = 256
_M = 2048
_D = 64
_H = 1024
_G = 16
_SC_CORES = 2
_SC_SUBCORES = 16
_BIG = 3.0e38


def _topk4_sc(usage):
    nw = _SC_CORES * _SC_SUBCORES
    bpw = _BS // nw
    mesh = plsc.VectorSubcoreMesh(core_axis_name="c", subcore_axis_name="s")

    @functools.partial(
        pl.kernel,
        out_type=jax.ShapeDtypeStruct((_BS * _NB,), jnp.int32),
        mesh=mesh,
        scratch_types=[
            pltpu.VMEM((_M,), jnp.float32),
            pltpu.VMEM((bpw * _NB,), jnp.int32),
        ],
    )
    def topk_kernel(u_hbm, out_hbm, u_buf, out_buf):
        wid = lax.axis_index("s") * _SC_CORES + lax.axis_index("c")
        lanes = lax.broadcasted_iota(jnp.int32, (16,), 0)
        r0 = jnp.zeros((16,), jnp.int32)
        r1 = jnp.zeros((16,), jnp.int32)
        for bi in range(bpw):
            batch = wid * bpw + bi
            pltpu.sync_copy(u_hbm.at[batch], u_buf)
            for p in range(_NB):
                def body(j, c):
                    mv, mi = c
                    v = u_buf[pl.ds(pl.multiple_of(j * 16, 16), 16)]
                    fi = j * 16 + lanes
                    take = v < mv
                    return (jnp.where(take, v, mv), jnp.where(take, fi, mi))

                mv, mi = lax.fori_loop(
                    0, _M // 16, body,
                    (jnp.full((16,), _BIG, jnp.float32),
                     jnp.zeros((16,), jnp.int32)))
                m = mv[0]
                i = mi[0]
                for l in range(1, 16):
                    v = mv[l]
                    ii = mi[l]
                    upd = (v < m) | ((v == m) & (ii < i))
                    m = jnp.where(upd, v, m)
                    i = jnp.where(upd, ii, i)
                pos = bi * _NB + p
                if pos < 16:
                    r0 = jnp.where(lanes == pos, i, r0)
                else:
                    r1 = jnp.where(lanes == (pos - 16), i, r1)
                blk = (i // 16) * 16
                sl = pl.ds(pl.multiple_of(blk, 16), 16)
                u_buf[sl] = jnp.where(lanes == (i - blk), _BIG, u_buf[sl])
        out_buf[pl.ds(0, 16)] = r0
        out_buf[pl.ds(16, 16)] = r1
        pltpu.sync_copy(
            out_buf, out_hbm.at[pl.ds(wid * (bpw * _NB), bpw * _NB)])

    return topk_kernel(usage)


def _query_kernel(h_ref, w_ref, bias_ref, q_ref, qss_ref):
    q = lax.dot_general(h_ref[...], w_ref[...], (((1,), (1,)), ((), ())),
                        preferred_element_type=jnp.float32) + bias_ref[...]
    q_ref[...] = q
    qss_ref[...] = jnp.full((1, 128), jnp.sum(q * q) * (1.0 / 128.0))


def _query_call(hidden, W, bias2):
    return pl.pallas_call(
        _query_kernel,
        out_shape=[
            jax.ShapeDtypeStruct((_BS, _NB * _D), jnp.float32),
            jax.ShapeDtypeStruct((1, 128), jnp.float32),
        ],
    )(hidden, W, bias2)


def _stage1_kernel(memt_ref, rw_ref, alpha_ref, idx_ref, q_ref,
                   mnewt_ref, logits_ref, wws_ref, mn2_ref):
    for g in range(_G):
        a = jax.nn.sigmoid(alpha_ref[g, :, 0])
        ww = a[:, None] * rw_ref[g]
        col = idx_ref[g, 0, :]
        hit = lax.broadcasted_iota(jnp.int32, (_NB, _M), 1) == col[:, None]
        ww = ww + jnp.where(hit, (1.0 - a)[:, None], 0.0)
        q = q_ref[g]
        deltat = lax.dot_general(q, ww, (((0,), (0,)), ((), ())),
                                 preferred_element_type=jnp.float32)
        mnt = memt_ref[g] + deltat
        mnewt_ref[g] = mnt
        logits_ref[g] = lax.dot_general(q, mnt, (((1,), (0,)), ((), ())),
                                        preferred_element_type=jnp.float32)
        wws_ref[g, 0, :] = jnp.sum(ww, axis=0)
        mn2_ref[g, 0, :] = jnp.full((128,), jnp.sum(mnt * mnt) * (1.0 / 128.0))


def _stage1_call(memt, read_weight, alpha, idx3, q3):
    return pl.pallas_call(
        _stage1_kernel,
        grid=(_BS // _G,),
        in_specs=[
            pl.BlockSpec((_G, _D, _M), lambda i: (i, 0, 0)),
            pl.BlockSpec((_G, _NB, _M), lambda i: (i, 0, 0)),
            pl.BlockSpec((_G, _NB, 1), lambda i: (i, 0, 0)),
            pl.BlockSpec((_G, 1, _NB), lambda i: (i, 0, 0)),
            pl.BlockSpec((_G, _NB, _D), lambda i: (i, 0, 0)),
        ],
        out_specs=[
            pl.BlockSpec((_G, _D, _M), lambda i: (i, 0, 0)),
            pl.BlockSpec((_G, _NB, _M), lambda i: (i, 0, 0)),
            pl.BlockSpec((_G, 1, _M), lambda i: (i, 0, 0)),
            pl.BlockSpec((_G, 1, 128), lambda i: (i, 0, 0)),
        ],
        out_shape=[
            jax.ShapeDtypeStruct((_BS, _D, _M), jnp.float32),
            jax.ShapeDtypeStruct((_BS, _NB, _M), jnp.float32),
            jax.ShapeDtypeStruct((_BS, 1, _M), jnp.float32),
            jax.ShapeDtypeStruct((_BS, 1, 128), jnp.float32),
        ],
    )(memt, read_weight, alpha, idx3, q3)


def _stage2_kernel(logits_ref, mnewt_ref, usage_ref, wws_ref, mn2_ref, qss_ref,
                   rw_ref, rv_ref, uw_ref):
    scale = jnp.sqrt(jnp.sum(mn2_ref[...])) * jnp.sqrt(jnp.sum(qss_ref[...]))
    inv = 1.0 / scale
    for g in range(_G):
        l = logits_ref[g] * inv
        m = jnp.max(l, axis=1, keepdims=True)
        e = jnp.exp(l - m)
        r = e / jnp.sum(e, axis=1, keepdims=True)
        rw_ref[g] = r
        rv_ref[g] = lax.dot_general(r, mnewt_ref[g], (((1,), (1,)), ((), ())),
                                    preferred_element_type=jnp.float32)
        uw_ref[g, 0, :] = (_GAMMA * usage_ref[g, 0, :] + jnp.sum(r, axis=0)
                           + wws_ref[g, 0, :])


def _stage2_call(logits, mnewt, usage3, wws, mn2, qss):
    return pl.pallas_call(
        _stage2_kernel,
        grid=(_BS // _G,),
        in_specs=[
            pl.BlockSpec((_G, _NB, _M), lambda i: (i, 0, 0)),
            pl.BlockSpec((_G, _D, _M), lambda i: (i, 0, 0)),
            pl.BlockSpec((_G, 1, _M), lambda i: (i, 0, 0)),
            pl.BlockSpec((_G, 1, _M), lambda i: (i, 0, 0)),
            pl.BlockSpec((_BS, 1, 128), lambda i: (0, 0, 0)),
            pl.BlockSpec((1, 128), lambda i: (0, 0)),
        ],
        out_specs=[
            pl.BlockSpec((_G, _NB, _M), lambda i: (i, 0, 0)),
            pl.BlockSpec((_G, _NB, _D), lambda i: (i, 0, 0)),
            pl.BlockSpec((_G, 1, _M), lambda i: (i, 0, 0)),
        ],
        out_shape=[
            jax.ShapeDtypeStruct((_BS, _NB, _M), jnp.float32),
            jax.ShapeDtypeStruct((_BS, _NB, _D), jnp.float32),
            jax.ShapeDtypeStruct((_BS, 1, _M), jnp.float32),
        ],
    )(logits, mnewt, usage3, wws, mn2, qss)


def kernel(memory, hidden, read_weight, usage_weight, alpha, W, b):
    idx = _topk4_sc(usage_weight)
    query, qss = _query_call(hidden, W, b.reshape(1, -1))
    idx3 = idx.reshape(_BS, 1, _NB)
    q3 = query.reshape(_BS, _NB, _D)
    memt = jnp.transpose(memory, (0, 2, 1))
    mnewt, logits, wws, mn2 = _stage1_call(memt, read_weight, alpha,
                                           idx3, q3)
    rw, rv, uw = _stage2_call(logits, mnewt,
                              usage_weight.reshape(_BS, 1, _M), wws, mn2, qss)
    memory_new = jnp.transpose(mnewt, (0, 2, 1))
    return (rv.reshape(_BS, _NB * _D), memory_new, rw, uw.reshape(_BS, _M))


# speedup vs baseline: 3.7001x; 1.1034x over previous
import functools

import jax
import jax.numpy as jnp
from jax import lax
from jax.experimental import pallas as pl
from jax.experimental.pallas import tpu as pltpu
from jax.experimental.pallas import tpu_sc as plsc

_NB = 4
_GAMMA = 0.95
_BS = 256
_M = 2048
_D = 64
_H = 1024
_G = 16
_SC_CORES = 2
_SC_SUBCORES = 16
_BIG = 3.0e38


def _topk4_sc(usage):
    nw = _SC_CORES * _SC_SUBCORES
    bpw = _BS // nw
    ns = 8
    ch = _M // 16 // ns
    mesh = plsc.VectorSubcoreMesh(core_axis_name="c", subcore_axis_name="s")

    @functools.partial(
        pl.kernel,
        out_type=jax.ShapeDtypeStruct((_BS * _NB,), jnp.int32),
        mesh=mesh,
        scratch_types=[
            pltpu.VMEM((_M,), jnp.float32),
            pltpu.VMEM((_M,), jnp.float32),
            pltpu.VMEM((bpw * _NB,), jnp.int32),
            pltpu.SemaphoreType.DMA,
            pltpu.SemaphoreType.DMA,
        ],
    )
    def topk_kernel(u_hbm, out_hbm, u_buf0, u_buf1, out_buf, sem0, sem1):
        wid = lax.axis_index("s") * _SC_CORES + lax.axis_index("c")
        lanes = lax.broadcasted_iota(jnp.int32, (16,), 0)
        r0 = jnp.zeros((16,), jnp.int32)
        r1 = jnp.zeros((16,), jnp.int32)
        bufs = [u_buf0, u_buf1]
        sems = [sem0, sem1]
        base = wid * bpw
        cp = pltpu.async_copy(u_hbm.at[base], u_buf0, sem0)
        for bi in range(bpw):
            u_buf = bufs[bi % 2]
            cp.wait()
            if bi + 1 < bpw:
                cp = pltpu.async_copy(u_hbm.at[base + bi + 1],
                                      bufs[(bi + 1) % 2], sems[(bi + 1) % 2])
            for p in range(_NB):
                def body(j, c):
                    new = []
                    for s in range(ns):
                        mv, mi = c[2 * s], c[2 * s + 1]
                        v = u_buf[pl.ds(
                            pl.multiple_of(s * (ch * 16) + j * 16, 16), 16)]
                        fi = (s * ch + j) * 16 + lanes
                        take = v < mv
                        new.append(jnp.where(take, v, mv))
                        new.append(jnp.where(take, fi, mi))
                    return tuple(new)

                init = []
                for s in range(ns):
                    init += [jnp.full((16,), _BIG, jnp.float32),
                             jnp.zeros((16,), jnp.int32)]
                c = lax.fori_loop(0, ch, body, tuple(init))
                vs = [c[2 * s] for s in range(ns)]
                ix = [c[2 * s + 1] for s in range(ns)]
                while len(vs) > 1:
                    nv, ni = [], []
                    for k in range(0, len(vs), 2):
                        t = ((vs[k + 1] < vs[k])
                             | ((vs[k + 1] == vs[k]) & (ix[k + 1] < ix[k])))
                        nv.append(jnp.where(t, vs[k + 1], vs[k]))
                        ni.append(jnp.where(t, ix[k + 1], ix[k]))
                    vs, ix = nv, ni
                mv, mi = vs[0], ix[0]
                m = mv[0]
                i = mi[0]
                for l in range(1, 16):
                    v = mv[l]
                    ii = mi[l]
                    upd = (v < m) | ((v == m) & (ii < i))
                    m = jnp.where(upd, v, m)
                    i = jnp.where(upd, ii, i)
                pos = bi * _NB + p
                if pos < 16:
                    r0 = jnp.where(lanes == pos, i, r0)
                else:
                    r1 = jnp.where(lanes == (pos - 16), i, r1)
                blk = (i // 16) * 16
                sl = pl.ds(pl.multiple_of(blk, 16), 16)
                u_buf[sl] = jnp.where(lanes == (i - blk), _BIG, u_buf[sl])
        out_buf[pl.ds(0, 16)] = r0
        out_buf[pl.ds(16, 16)] = r1
        pltpu.sync_copy(
            out_buf, out_hbm.at[pl.ds(wid * (bpw * _NB), bpw * _NB)])

    return topk_kernel(usage)


def _query_kernel(h_ref, w_ref, bias_ref, q_ref, qss_ref):
    q = lax.dot_general(h_ref[...], w_ref[...], (((1,), (1,)), ((), ())),
                        preferred_element_type=jnp.float32) + bias_ref[...]
    q_ref[...] = q
    qss_ref[...] = jnp.full((1, 128), jnp.sum(q * q) * (1.0 / 128.0))


def _query_call(hidden, W, bias2):
    return pl.pallas_call(
        _query_kernel,
        out_shape=[
            jax.ShapeDtypeStruct((_BS, _NB * _D), jnp.float32),
            jax.ShapeDtypeStruct((1, 128), jnp.float32),
        ],
    )(hidden, W, bias2)


def _stage1_kernel(memt_ref, rw_ref, alpha_ref, idx_ref, q_ref,
                   mnewt_ref, logits_ref, wws_ref, mn2_ref):
    for g in range(_G):
        a = jax.nn.sigmoid(alpha_ref[g, :, 0])
        ww = a[:, None] * rw_ref[g]
        col = idx_ref[g, 0, :]
        hit = lax.broadcasted_iota(jnp.int32, (_NB, _M), 1) == col[:, None]
        ww = ww + jnp.where(hit, (1.0 - a)[:, None], 0.0)
        q = q_ref[g]
        deltat = lax.dot_general(q, ww, (((0,), (0,)), ((), ())),
                                 preferred_element_type=jnp.float32)
        mnt = memt_ref[g] + deltat
        mnewt_ref[g] = mnt
        logits_ref[g] = lax.dot_general(q, mnt, (((1,), (0,)), ((), ())),
                                        preferred_element_type=jnp.float32)
        wws_ref[g, 0, :] = jnp.sum(ww, axis=0)
        mn2_ref[g, 0, :] = jnp.full((128,), jnp.sum(mnt * mnt) * (1.0 / 128.0))


def _stage1_call(memt, read_weight, alpha, idx3, q3):
    return pl.pallas_call(
        _stage1_kernel,
        grid=(_BS // _G,),
        in_specs=[
            pl.BlockSpec((_G, _D, _M), lambda i: (i, 0, 0)),
            pl.BlockSpec((_G, _NB, _M), lambda i: (i, 0, 0)),
            pl.BlockSpec((_G, _NB, 1), lambda i: (i, 0, 0)),
            pl.BlockSpec((_G, 1, _NB), lambda i: (i, 0, 0)),
            pl.BlockSpec((_G, _NB, _D), lambda i: (i, 0, 0)),
        ],
        out_specs=[
            pl.BlockSpec((_G, _D, _M), lambda i: (i, 0, 0)),
            pl.BlockSpec((_G, _NB, _M), lambda i: (i, 0, 0)),
            pl.BlockSpec((_G, 1, _M), lambda i: (i, 0, 0)),
            pl.BlockSpec((_G, 1, 128), lambda i: (i, 0, 0)),
        ],
        out_shape=[
            jax.ShapeDtypeStruct((_BS, _D, _M), jnp.float32),
            jax.ShapeDtypeStruct((_BS, _NB, _M), jnp.float32),
            jax.ShapeDtypeStruct((_BS, 1, _M), jnp.float32),
            jax.ShapeDtypeStruct((_BS, 1, 128), jnp.float32),
        ],
    )(memt, read_weight, alpha, idx3, q3)


def _stage2_kernel(logits_ref, mnewt_ref, usage_ref, wws_ref, mn2_ref, qss_ref,
                   rw_ref, rv_ref, uw_ref):
    scale = jnp.sqrt(jnp.sum(mn2_ref[...])) * jnp.sqrt(jnp.sum(qss_ref[...]))
    inv = 1.0 / scale
    for g in range(_G):
        l = logits_ref[g] * inv
        m = jnp.max(l, axis=1, keepdims=True)
        e = jnp.exp(l - m)
        r = e / jnp.sum(e, axis=1, keepdims=True)
        rw_ref[g] = r
        rv_ref[g] = lax.dot_general(r, mnewt_ref[g], (((1,), (1,)), ((), ())),
                                    preferred_element_type=jnp.float32)
        uw_ref[g, 0, :] = (_GAMMA * usage_ref[g, 0, :] + jnp.sum(r, axis=0)
                           + wws_ref[g, 0, :])


def _stage2_call(logits, mnewt, usage3, wws, mn2, qss):
    return pl.pallas_call(
        _stage2_kernel,
        grid=(_BS // _G,),
        in_specs=[
            pl.BlockSpec((_G, _NB, _M), lambda i: (i, 0, 0)),
            pl.BlockSpec((_G, _D, _M), lambda i: (i, 0, 0)),
            pl.BlockSpec((_G, 1, _M), lambda i: (i, 0, 0)),
            pl.BlockSpec((_G, 1, _M), lambda i: (i, 0, 0)),
            pl.BlockSpec((_BS, 1, 128), lambda i: (0, 0, 0)),
            pl.BlockSpec((1, 128), lambda i: (0, 0)),
        ],
        out_specs=[
            pl.BlockSpec((_G, _NB, _M), lambda i: (i, 0, 0)),
            pl.BlockSpec((_G, _NB, _D), lambda i: (i, 0, 0)),
            pl.BlockSpec((_G, 1, _M), lambda i: (i, 0, 0)),
        ],
        out_shape=[
            jax.ShapeDtypeStruct((_BS, _NB, _M), jnp.float32),
            jax.ShapeDtypeStruct((_BS, _NB, _D), jnp.float32),
            jax.ShapeDtypeStruct((_BS, 1, _M), jnp.float32),
        ],
    )(logits, mnewt, usage3, wws, mn2, qss)


def kernel(memory, hidden, read_weight, usage_weight, alpha, W, b):
    idx = _topk4_sc(usage_weight)
    query, qss = _query_call(hidden, W, b.reshape(1, -1))
    idx3 = idx.reshape(_BS, 1, _NB)
    q3 = query.reshape(_BS, _NB, _D)
    memt = jnp.transpose(memory, (0, 2, 1))
    mnewt, logits, wws, mn2 = _stage1_call(memt, read_weight, alpha,
                                           idx3, q3)
    rw, rv, uw = _stage2_call(logits, mnewt,
                              usage_weight.reshape(_BS, 1, _M), wws, mn2, qss)
    memory_new = jnp.transpose(mnewt, (0, 2, 1))
    return (rv.reshape(_BS, _NB * _D), memory_new, rw, uw.reshape(_BS, _M))
